# baseline (speedup 1.0000x reference)
import functools

import jax
import jax.numpy as jnp
from jax import lax
from jax.experimental import pallas as pl
from jax.experimental.pallas import tpu as pltpu
from jax.experimental.pallas import tpu_sc as plsc

_B = 16384
_ED = 32
_NS = 16
_BPW = _B // _NS
_HALF = 500000

_NUM_GENDERS = 2
_NUM_OCC = 21
_NUM_GENRES = 18
_H = 128

_TB = 2048


def _sc_gather2_body(uidx_hbm, midx_hbm, utab_hbm, mtab_hbm,
                     uemb_hbm, memb_hbm,
                     uidx_v, midx_v, urows_v, mrows_v, sem_u, sem_m):
    wid = lax.axis_index("s")
    base = wid * _BPW
    pltpu.sync_copy(uidx_hbm.at[pl.ds(base, _BPW)], uidx_v)
    pltpu.sync_copy(midx_hbm.at[pl.ds(base, _BPW)], midx_v)
    cu = pltpu.async_copy(utab_hbm.at[uidx_v], urows_v, sem_u)
    cm = pltpu.async_copy(mtab_hbm.at[midx_v], mrows_v, sem_m)
    cu.wait()
    cm.wait()
    pltpu.sync_copy(urows_v, uemb_hbm.at[pl.ds(base, _BPW)])
    pltpu.sync_copy(mrows_v, memb_hbm.at[pl.ds(base, _BPW)])


def _sc_gather1_body(uidx_hbm, utab_hbm, uemb_hbm, uidx_v, urows_v, sem_u):
    wid = lax.axis_index("s")
    base = wid * _BPW
    pltpu.sync_copy(uidx_hbm.at[pl.ds(base, _BPW)], uidx_v)
    pltpu.async_copy(utab_hbm.at[uidx_v], urows_v, sem_u).wait()
    pltpu.sync_copy(urows_v, uemb_hbm.at[pl.ds(base, _BPW)])


_PARAMS = dict(
    compiler_params=pltpu.CompilerParams(use_tc_tiling_on_sc=False),
)


@functools.cache
def _sc_gather2():
    return pl.kernel(
        _sc_gather2_body,
        out_type=(jax.ShapeDtypeStruct((_B, _ED), jnp.float32),
                  jax.ShapeDtypeStruct((_B, _ED), jnp.float32)),
        mesh=plsc.VectorSubcoreMesh(core_axis_name="c", subcore_axis_name="s",
                                    num_cores=1, num_subcores=_NS),
        scratch_types=[
            pltpu.VMEM((_BPW,), jnp.int32),
            pltpu.VMEM((_BPW,), jnp.int32),
            pltpu.VMEM((_BPW, _ED), jnp.float32),
            pltpu.VMEM((_BPW, _ED), jnp.float32),
            pltpu.SemaphoreType.DMA,
            pltpu.SemaphoreType.DMA,
        ],
        **_PARAMS,
    )


@functools.cache
def _sc_gather1():
    return pl.kernel(
        _sc_gather1_body,
        out_type=jax.ShapeDtypeStruct((_B, _ED), jnp.float32),
        mesh=plsc.VectorSubcoreMesh(core_axis_name="c", subcore_axis_name="s",
                                    num_cores=1, num_subcores=_NS),
        scratch_types=[
            pltpu.VMEM((_BPW,), jnp.int32),
            pltpu.VMEM((_BPW, _ED), jnp.float32),
            pltpu.SemaphoreType.DMA,
        ],
        **_PARAMS,
    )


def _mlp_body(uembA, uembB, memb, user, gender, occ, genres, gtab, otab,
              wg, bg, w1, b1, w2, b2, out):
    w1r = w1[...]
    f32 = jnp.float32
    genre_w = jnp.dot(wg[...], w1r[128:160, :], preferred_element_type=f32)
    gt_w = jnp.dot(gtab[...], w1r[64:96, :], preferred_element_type=f32)
    ot_w = jnp.dot(otab[...], w1r[96:128, :], preferred_element_type=f32)
    bias = b1[...] + jnp.dot(bg[...], w1r[128:160, :], preferred_element_type=f32)

    uemb = jnp.where(user[...] < _HALF, uembA[...], uembB[...])

    g1h = (lax.broadcasted_iota(jnp.int32, (_TB, _NUM_GENDERS), 1)
           == gender[...]).astype(f32)
    o1h = (lax.broadcasted_iota(jnp.int32, (_TB, _NUM_OCC), 1)
           == occ[...]).astype(f32)

    h = (bias
         + jnp.dot(uemb, w1r[0:32, :], preferred_element_type=f32)
         + jnp.dot(memb[...], w1r[32:64, :], preferred_element_type=f32)
         + jnp.dot(g1h, gt_w, preferred_element_type=f32)
         + jnp.dot(o1h, ot_w, preferred_element_type=f32)
         + jnp.dot(genres[...], genre_w, preferred_element_type=f32))
    h = jnp.maximum(h, 0.0)
    out[...] = jnp.dot(h, w2[...], preferred_element_type=f32) + b2[...]


def _mlp_call(uembA, uembB, memb, user2d, gender2d, occ2d, genres,
              gtab, otab, wg, bg2d, w1, b12d, w2, b22d):
    grid = (_B // _TB,)
    full = lambda i: (0, 0)
    return pl.pallas_call(
        _mlp_body,
        grid=grid,
        in_specs=[
            pl.BlockSpec((_TB, _ED), lambda i: (i, 0)),
            pl.BlockSpec((_TB, _ED), lambda i: (i, 0)),
            pl.BlockSpec((_TB, _ED), lambda i: (i, 0)),
            pl.BlockSpec((_TB, 1), lambda i: (i, 0)),
            pl.BlockSpec((_TB, 1), lambda i: (i, 0)),
            pl.BlockSpec((_TB, 1), lambda i: (i, 0)),
            pl.BlockSpec((_TB, _NUM_GENRES), lambda i: (i, 0)),
            pl.BlockSpec((_NUM_GENDERS, _ED), full),
            pl.BlockSpec((_NUM_OCC, _ED), full),
            pl.BlockSpec((_NUM_GENRES, _ED), full),
            pl.BlockSpec((1, _ED), full),
            pl.BlockSpec((5 * _ED, _H), full),
            pl.BlockSpec((1, _H), full),
            pl.BlockSpec((_H, 1), full),
            pl.BlockSpec((1, 1), full),
        ],
        out_specs=pl.BlockSpec((_TB, 1), lambda i: (i, 0)),
        out_shape=jax.ShapeDtypeStruct((_B, 1), jnp.float32),
    )(uembA, uembB, memb, user2d, gender2d, occ2d, genres,
      gtab, otab, wg, bg2d, w1, b12d, w2, b22d)


def kernel(user, movie, gender, occupation, genres,
           user_table, movie_table, gender_table, occupation_table,
           W_genre, b_genre, W1, b1, W2, b2):
    user = user.astype(jnp.int32)
    movie = movie.astype(jnp.int32)
    in_lo = user < _HALF
    uidxA = jnp.where(in_lo, user, 0)
    uidxB = jnp.where(in_lo, 0, user - _HALF)
    uembA, memb = _sc_gather2()(uidxA, movie, user_table[:_HALF], movie_table)
    uembB = _sc_gather1()(uidxB, user_table[_HALF:])
    out = _mlp_call(
        uembA, uembB, memb,
        user.reshape(_B, 1),
        gender.astype(jnp.int32).reshape(_B, 1),
        occupation.astype(jnp.int32).reshape(_B, 1),
        genres.astype(jnp.float32),
        gender_table, occupation_table,
        W_genre, b_genre.reshape(1, _ED),
        W1, b1.reshape(1, _H), W2, b2.reshape(1, 1),
    )
    return out.reshape(_B)


# speedup vs baseline: 1.2499x; 1.2499x over previous
import functools

import jax
import jax.numpy as jnp
from jax import lax
from jax.experimental import pallas as pl
from jax.experimental.pallas import tpu as pltpu
from jax.experimental.pallas import tpu_sc as plsc

_B = 16384
_ED = 32
_NC = 2
_NS = 16
_NW = _NC * _NS
_BPW = _B // _NW
_CH = 128
_NCH = _BPW // _CH

_NUM_GENDERS = 2
_NUM_OCC = 21
_NUM_GENRES = 18
_H = 128

_TB = 2048
_PB = 1000


def _pack_body(tab, out):
    pairs = tab[...].astype(jnp.bfloat16).reshape(_PB, 2, 128)
    bits = lax.bitcast_convert_type(pairs, jnp.int16)
    lo = bits[:, 0, :].astype(jnp.int32) & 0xFFFF
    hi = bits[:, 1, :].astype(jnp.int32) << 16
    out[...] = hi | lo


def _pack_call(table4):
    step = 2 * _PB
    if table4.shape[0] % step:
        table4 = jnp.pad(
            table4, ((0, step - table4.shape[0] % step), (0, 0)))
    n = table4.shape[0] // 2
    return pl.pallas_call(
        _pack_body,
        grid=(n // _PB,),
        in_specs=[pl.BlockSpec((2 * _PB, 128), lambda i: (i, 0))],
        out_specs=pl.BlockSpec((_PB, 128), lambda i: (i, 0)),
        out_shape=jax.ShapeDtypeStruct((n, 128), jnp.int32),
    )(table4)


def _sc_gather_body(uidx_hbm, midx_hbm, utab_hbm, mtab_hbm,
                    uemb_hbm, memb_hbm,
                    uidx_v, midx_v, urows_v, mrows_v, sem_u, sem_m):
    wid = lax.axis_index("s") * _NC + lax.axis_index("c")
    base = wid * _NCH
    pltpu.sync_copy(uidx_hbm.at[pl.ds(base, _NCH)], uidx_v)
    pltpu.sync_copy(midx_hbm.at[pl.ds(base, _NCH)], midx_v)
    for c in range(_NCH):
        cu = pltpu.async_copy(utab_hbm.at[uidx_v.at[c]], urows_v, sem_u)
        cm = pltpu.async_copy(mtab_hbm.at[midx_v.at[c]], mrows_v, sem_m)
        cu.wait()
        cm.wait()
        row0 = (base + c) * _CH
        pltpu.sync_copy(urows_v, uemb_hbm.at[pl.ds(row0, _CH)])
        pltpu.sync_copy(mrows_v, memb_hbm.at[pl.ds(row0, _CH)])


@functools.cache
def _sc_gather():
    return pl.kernel(
        _sc_gather_body,
        out_type=(jax.ShapeDtypeStruct((_B, 128), jnp.int32),
                  jax.ShapeDtypeStruct((_B, 128), jnp.int32)),
        mesh=plsc.VectorSubcoreMesh(core_axis_name="c", subcore_axis_name="s",
                                    num_cores=_NC, num_subcores=_NS),
        scratch_types=[
            pltpu.VMEM((_NCH, _CH), jnp.int32),
            pltpu.VMEM((_NCH, _CH), jnp.int32),
            pltpu.VMEM((_CH, 128), jnp.int32),
            pltpu.VMEM((_CH, 128), jnp.int32),
            pltpu.SemaphoreType.DMA,
            pltpu.SemaphoreType.DMA,
        ],
    )


def _unpack_selected(pack, sub):
    word_grp = lax.broadcasted_iota(jnp.int32, (_TB, 128), 1) >> 5
    mask = word_grp == (sub >> 1)
    even = (sub & 1) == 0
    w_even = jnp.where(mask & even, pack, 0)
    w_odd = jnp.where(mask & (~even), pack, 0)
    lo = lax.bitcast_convert_type(
        jnp.left_shift(w_even & 0xFFFF, 16), jnp.float32)
    hi = lax.bitcast_convert_type(w_odd & jnp.int32(-65536), jnp.float32)
    return lo + hi


def _mlp_body(upack, mpack, user, movie, gender, occ, genres, gtab, otab,
              wg, bg, w1u4, w1m4, w1, b1, w2, b2, out):
    w1r = w1[...]
    f32 = jnp.float32
    genre_w = jnp.dot(wg[...], w1r[128:160, :], preferred_element_type=f32)
    gt_w = jnp.dot(gtab[...], w1r[64:96, :], preferred_element_type=f32)
    ot_w = jnp.dot(otab[...], w1r[96:128, :], preferred_element_type=f32)
    bias = b1[...] + jnp.dot(bg[...], w1r[128:160, :], preferred_element_type=f32)

    uval = _unpack_selected(upack[...], user[...])
    mval = _unpack_selected(mpack[...], movie[...])

    g1h = (lax.broadcasted_iota(jnp.int32, (_TB, _NUM_GENDERS), 1)
           == gender[...]).astype(f32)
    o1h = (lax.broadcasted_iota(jnp.int32, (_TB, _NUM_OCC), 1)
           == occ[...]).astype(f32)

    h = (bias
         + jnp.dot(uval, w1u4[...], preferred_element_type=f32)
         + jnp.dot(mval, w1m4[...], preferred_element_type=f32)
         + jnp.dot(g1h, gt_w, preferred_element_type=f32)
         + jnp.dot(o1h, ot_w, preferred_element_type=f32)
         + jnp.dot(genres[...], genre_w, preferred_element_type=f32))
    h = jnp.maximum(h, 0.0)
    out[...] = jnp.dot(h, w2[...], preferred_element_type=f32) + b2[...]


def _mlp_call(upack, mpack, usub2d, msub2d, gender2d, occ2d, genres,
              gtab, otab, wg, bg2d, w1u4, w1m4, w1, b12d, w2, b22d):
    grid = (_B // _TB,)
    full = lambda i: (0, 0)
    return pl.pallas_call(
        _mlp_body,
        grid=grid,
        in_specs=[
            pl.BlockSpec((_TB, 128), lambda i: (i, 0)),
            pl.BlockSpec((_TB, 128), lambda i: (i, 0)),
            pl.BlockSpec((_TB, 1), lambda i: (i, 0)),
            pl.BlockSpec((_TB, 1), lambda i: (i, 0)),
            pl.BlockSpec((_TB, 1), lambda i: (i, 0)),
            pl.BlockSpec((_TB, 1), lambda i: (i, 0)),
            pl.BlockSpec((_TB, _NUM_GENRES), lambda i: (i, 0)),
            pl.BlockSpec((_NUM_GENDERS, _ED), full),
            pl.BlockSpec((_NUM_OCC, _ED), full),
            pl.BlockSpec((_NUM_GENRES, _ED), full),
            pl.BlockSpec((1, _ED), full),
            pl.BlockSpec((128, _H), full),
            pl.BlockSpec((128, _H), full),
            pl.BlockSpec((5 * _ED, _H), full),
            pl.BlockSpec((1, _H), full),
            pl.BlockSpec((_H, 1), full),
            pl.BlockSpec((1, 1), full),
        ],
        out_specs=pl.BlockSpec((_TB, 1), lambda i: (i, 0)),
        out_shape=jax.ShapeDtypeStruct((_B, 1), jnp.float32),
    )(upack, mpack, usub2d, msub2d, gender2d, occ2d, genres,
      gtab, otab, wg, bg2d, w1u4, w1m4, w1, b12d, w2, b22d)


def kernel(user, movie, gender, occupation, genres,
           user_table, movie_table, gender_table, occupation_table,
           W_genre, b_genre, W1, b1, W2, b2):
    user = user.astype(jnp.int32)
    movie = movie.astype(jnp.int32)
    usub = ((user & 3) << 1) | ((user >> 2) & 1)
    msub = ((movie & 3) << 1) | ((movie >> 2) & 1)
    upack, mpack = _sc_gather()(
        (user >> 3).reshape(_B // _CH, _CH),
        (movie >> 3).reshape(_B // _CH, _CH),
        _pack_call(user_table.reshape(-1, 128)),
        _pack_call(movie_table.reshape(-1, 128)))
    w1u4 = jnp.concatenate([W1[0:32]] * 4, axis=0)
    w1m4 = jnp.concatenate([W1[32:64]] * 4, axis=0)
    out = _mlp_call(
        upack, mpack,
        usub.reshape(_B, 1), msub.reshape(_B, 1),
        gender.astype(jnp.int32).reshape(_B, 1),
        occupation.astype(jnp.int32).reshape(_B, 1),
        genres.astype(jnp.float32),
        gender_table, occupation_table,
        W_genre, b_genre.reshape(1, _ED),
        w1u4, w1m4,
        W1, b1.reshape(1, _H), W2, b2.reshape(1, 1),
    )
    return out.reshape(_B)
